# baseline (speedup 1.0000x reference)
import jax
import jax.numpy as jnp
from jax import lax
from jax.experimental import pallas as pl
from jax.experimental.pallas import tpu as pltpu

N = 65536
C = 1000
KEEP = int(N * 0.7)
BLK = 1024
GRID = N // BLK
NSTREAM = 4
H = GRID // NSTREAM


def _fused_body(*refs):
    ins = refs[:NSTREAM]
    lbls = refs[NSTREAM:2 * NSTREAM]
    out_ref = refs[2 * NSTREAM]
    scratch = refs[2 * NSTREAM + 1]
    j = pl.program_id(0)
    for off in range(NSTREAM):
        x = ins[off][...]
        lbl = lbls[off][0, 0, :]
        s = jnp.sum(jnp.exp(x), axis=1)
        col = lax.broadcasted_iota(jnp.int32, (BLK, C), 1)
        pick = jnp.sum(jnp.where(col == lbl[:, None], x, 0.0), axis=1)
        loss = jnp.log(s) - pick
        scratch[pl.ds(j + off * H, 1), :, :] = loss.reshape(1, 1, BLK)

    @pl.when(j == H - 1)
    def _():
        x = scratch[...]
        b = lax.bitcast_convert_type(x, jnp.int32)
        m = jnp.where(b < 0, b ^ jnp.int32(0x7FFFFFFF), b)
        k = jnp.int32(KEEP)
        sign = jnp.int32(-0x80000000)

        def step(i, tb):
            bit = lax.shift_left(jnp.int32(1), jnp.int32(31) - i)
            cand = tb | bit
            c = jnp.sum((m >= (cand ^ sign)).astype(jnp.int32))
            return jnp.where(c >= k, cand, tb)

        tb = lax.fori_loop(0, 32, step, jnp.int32(0))
        t_signed = tb ^ sign
        t_bits = jnp.where(
            t_signed < 0, t_signed ^ jnp.int32(0x7FFFFFFF), t_signed)
        t = lax.bitcast_convert_type(t_bits, jnp.float32)
        gt = m > t_signed
        s_gt = jnp.sum(jnp.where(gt, x, 0.0))
        c_gt = jnp.sum(gt.astype(jnp.int32))
        out_ref[0, 0] = (
            s_gt + (k - c_gt).astype(jnp.float32) * t) / jnp.float32(KEEP)


def kernel(class_out, label):
    lbl = label.astype(jnp.int32).reshape(GRID, 1, BLK)

    def mkx(off):
        return pl.BlockSpec((BLK, C), lambda i, off=off: (i + off * H, 0))

    def mkl(off):
        return pl.BlockSpec((1, 1, BLK), lambda i, off=off: (i + off * H, 0, 0))

    res = pl.pallas_call(
        _fused_body,
        grid=(H,),
        in_specs=[mkx(j) for j in range(NSTREAM)]
        + [mkl(j) for j in range(NSTREAM)],
        out_specs=pl.BlockSpec(memory_space=pltpu.SMEM),
        out_shape=jax.ShapeDtypeStruct((1, 1), jnp.float32),
        scratch_shapes=[pltpu.VMEM((GRID, 1, BLK), jnp.float32)],
        compiler_params=pltpu.CompilerParams(
            dimension_semantics=("arbitrary",)),
    )(*([class_out] * NSTREAM + [lbl] * NSTREAM))
    return res[0, 0]


# speedup vs baseline: 1.0242x; 1.0242x over previous
import jax
import jax.numpy as jnp
from jax import lax
from jax.experimental import pallas as pl
from jax.experimental.pallas import tpu as pltpu

N = 65536
C = 1000
KEEP = int(N * 0.7)
BLK = 1024
GRID = N // BLK
NSTREAM = 4
H = GRID // NSTREAM


def _loss_body(*refs):
    ins = refs[:NSTREAM]
    lbls = refs[NSTREAM:2 * NSTREAM]
    outs = refs[2 * NSTREAM:]
    for j in range(NSTREAM):
        x = ins[j][...]
        lbl = lbls[j][0, 0, :]
        s = jnp.sum(jnp.exp(x), axis=1)
        col = lax.broadcasted_iota(jnp.int32, (BLK, C), 1)
        pick = jnp.sum(jnp.where(col == lbl[:, None], x, 0.0), axis=1)
        outs[j][0, 0, :] = jnp.log(s) - pick


def _topk_mean_body(*refs):
    parts = [r[...].reshape(H * BLK // 128, 128) for r in refs[:NSTREAM]]
    out_ref = refs[NSTREAM]
    x = jnp.concatenate(parts, axis=0)
    b = lax.bitcast_convert_type(x, jnp.int32)
    m = jnp.where(b < 0, b ^ jnp.int32(0x7FFFFFFF), b)
    k = jnp.int32(KEEP)
    sign = jnp.int32(-0x80000000)

    def step(i, tb):
        bit = lax.shift_left(jnp.int32(1), jnp.int32(31) - i)
        cand = tb | bit
        c = jnp.sum((m >= (cand ^ sign)).astype(jnp.int32))
        return jnp.where(c >= k, cand, tb)

    tb = lax.fori_loop(0, 32, step, jnp.int32(0))
    t_signed = tb ^ sign
    t_bits = jnp.where(t_signed < 0, t_signed ^ jnp.int32(0x7FFFFFFF), t_signed)
    t = lax.bitcast_convert_type(t_bits, jnp.float32)
    gt = m > t_signed
    s_gt = jnp.sum(jnp.where(gt, x, 0.0))
    c_gt = jnp.sum(gt.astype(jnp.int32))
    out_ref[0, 0] = (s_gt + (k - c_gt).astype(jnp.float32) * t) / jnp.float32(KEEP)


def kernel(class_out, label):
    lbl = label.astype(jnp.int32).reshape(GRID, 1, BLK)

    def mkx(off):
        return pl.BlockSpec((BLK, C), lambda i, off=off: (i + off * H, 0))

    def mkl(off):
        return pl.BlockSpec((1, 1, BLK), lambda i, off=off: (i + off * H, 0, 0))

    parts = pl.pallas_call(
        _loss_body,
        grid=(H,),
        in_specs=[mkx(j) for j in range(NSTREAM)]
        + [mkl(j) for j in range(NSTREAM)],
        out_specs=[pl.BlockSpec((1, 1, BLK), lambda i: (i, 0, 0))] * NSTREAM,
        out_shape=[jax.ShapeDtypeStruct((H, 1, BLK), jnp.float32)] * NSTREAM,
    )(*([class_out] * NSTREAM + [lbl] * NSTREAM))

    res = pl.pallas_call(
        _topk_mean_body,
        in_specs=[pl.BlockSpec((H, 1, BLK), lambda: (0, 0, 0))] * NSTREAM,
        out_specs=pl.BlockSpec(memory_space=pltpu.SMEM),
        out_shape=jax.ShapeDtypeStruct((1, 1), jnp.float32),
    )(*parts)
    return res[0, 0]
